# baseline (speedup 1.0000x reference)
import functools

import jax
import jax.numpy as jnp
from jax import lax
from jax.experimental import pallas as pl
from jax.experimental.pallas import tpu as pltpu
from jax.experimental.pallas import tpu_sc as plsc

DIM = 2048
HALF = 1024
NW = 32
P_PER_W = 128
CHUNK = 8
N_CHUNKS = P_PER_W // CHUNK
SLICES = DIM // 16


def _sc_body(x_hbm, yang_hbm, yin_hbm, pe_hbm, out_hbm,
             yang_v, yin_v, sig_v, xv0, xv1, xv2, xv3,
             in0, in1, in2, in3, out0, out1, out2, out3):
    c = lax.axis_index("c")
    s = lax.axis_index("s")
    wid = s * 2 + c
    s0 = wid * P_PER_W

    xvs = (xv0, xv1, xv2, xv3)
    ins = (in0, in1, in2, in3)
    outs = (out0, out1, out2, out3)

    pltpu.sync_copy(yang_hbm, yang_v)
    pltpu.sync_copy(yin_hbm, yin_v)

    def do_chunk(m, _):
        base = s0 + m * CHUNK

        for b in range(4):
            @pl.when(m > 0)
            def _(b=b):
                prev = base - CHUNK
                pltpu.make_async_copy(
                    xvs[b], out_hbm.at[b, pl.ds(prev, CHUNK)], outs[b]).wait()
            pltpu.async_copy(x_hbm.at[b, pl.ds(base, CHUNK)], xvs[b], ins[b])

        pltpu.sync_copy(pe_hbm.at[pl.ds(base, CHUNK)], sig_v)

        for j in range(CHUNK):
            r = lax.rem(base + j, 12)
            r6 = lax.rem(base + j + 6, 12)

            @plsc.parallel_loop(0, HALF // 16, unroll=16)
            def _(k, j=j, r=r, r6=r6):
                off = k * 16
                plsc.addupdate(sig_v.at[j, pl.ds(off, 16)],
                               yang_v[r, pl.ds(off, 16)])
                plsc.addupdate(sig_v.at[j, pl.ds(HALF + off, 16)],
                               yin_v[r6, pl.ds(off, 16)])

        for b in range(4):
            pltpu.make_async_copy(
                x_hbm.at[b, pl.ds(base, CHUNK)], xvs[b], ins[b]).wait()

        @plsc.parallel_loop(0, CHUNK * SLICES, unroll=8)
        def _(t):
            j = lax.shift_right_logical(t, 7)
            off = pl.multiple_of(
                lax.shift_left(lax.bitwise_and(t, SLICES - 1), 4), 16)
            sv = sig_v[j, pl.ds(off, 16)]
            for b in range(4):
                xvs[b][j, pl.ds(off, 16)] = xvs[b][j, pl.ds(off, 16)] + sv

        for b in range(4):
            pltpu.async_copy(xvs[b], out_hbm.at[b, pl.ds(base, CHUNK)], outs[b])
        return 0

    lax.fori_loop(0, N_CHUNKS, do_chunk, 0)

    last = s0 + (N_CHUNKS - 1) * CHUNK
    for b in range(4):
        pltpu.make_async_copy(
            xvs[b], out_hbm.at[b, pl.ds(last, CHUNK)], outs[b]).wait()


def kernel(x, yang_wheel, yin_wheel, grand_cycle_pe):
    b, s, d = x.shape
    mesh = plsc.VectorSubcoreMesh(core_axis_name="c", subcore_axis_name="s")
    k = functools.partial(
        pl.kernel,
        mesh=mesh,
        out_type=jax.ShapeDtypeStruct((b, s, d), x.dtype),
        scratch_types=[
            pltpu.VMEM(yang_wheel.shape, jnp.float32),
            pltpu.VMEM(yin_wheel.shape, jnp.float32),
            pltpu.VMEM((CHUNK, DIM), jnp.float32),
            pltpu.VMEM((CHUNK, DIM), jnp.float32),
            pltpu.VMEM((CHUNK, DIM), jnp.float32),
            pltpu.VMEM((CHUNK, DIM), jnp.float32),
            pltpu.VMEM((CHUNK, DIM), jnp.float32),
            pltpu.SemaphoreType.DMA,
            pltpu.SemaphoreType.DMA,
            pltpu.SemaphoreType.DMA,
            pltpu.SemaphoreType.DMA,
            pltpu.SemaphoreType.DMA,
            pltpu.SemaphoreType.DMA,
            pltpu.SemaphoreType.DMA,
            pltpu.SemaphoreType.DMA,
        ],
    )(_sc_body)
    return k(x, yang_wheel, yin_wheel, grand_cycle_pe)


# speedup vs baseline: 1.6842x; 1.6842x over previous
import functools

import jax
import jax.numpy as jnp
from jax import lax
from jax.experimental import pallas as pl
from jax.experimental.pallas import tpu as pltpu
from jax.experimental.pallas import tpu_sc as plsc

DIM = 2048
HALF = 1024
S_TILE = 1024


def _gather_body(yang_hbm, yin_hbm, cyc_hbm, yang_v, yin_v, cyc_v):
    c = lax.axis_index("c")
    s = lax.axis_index("s")
    wid = s * 2 + c

    @pl.when(wid == 0)
    def _():
        pltpu.sync_copy(yang_hbm, yang_v)
        pltpu.sync_copy(yin_hbm, yin_v)
        for i in range(12):
            i6 = (i + 6) % 12

            @plsc.parallel_loop(0, HALF // 16, unroll=8)
            def _(k, i=i, i6=i6):
                off = k * 16
                cyc_v[i, pl.ds(off, 16)] = yang_v[i, pl.ds(off, 16)]
                cyc_v[i, pl.ds(HALF + off, 16)] = yin_v[i6, pl.ds(off, 16)]

        pltpu.sync_copy(cyc_v, cyc_hbm)


def _sc_gather_cyc(yang_wheel, yin_wheel):
    mesh = plsc.VectorSubcoreMesh(core_axis_name="c", subcore_axis_name="s",
                                  num_cores=1, num_subcores=1)
    k = functools.partial(
        pl.kernel,
        mesh=mesh,
        out_type=jax.ShapeDtypeStruct((12, DIM), jnp.float32),
        scratch_types=[
            pltpu.VMEM((12, HALF), jnp.float32),
            pltpu.VMEM((12, HALF), jnp.float32),
            pltpu.VMEM((12, DIM), jnp.float32),
        ],
    )(_gather_body)
    return k(yang_wheel, yin_wheel)


def _enc_kernel(x_ref, cyc_ref, pe_ref, o_ref):
    i = pl.program_id(0)
    base = i * S_TILE
    pos = base + jax.lax.broadcasted_iota(jnp.int32, (S_TILE, 12), 0)
    col = jax.lax.broadcasted_iota(jnp.int32, (S_TILE, 12), 1)
    onehot = (pos % 12 == col).astype(jnp.float32)
    sig = jnp.dot(onehot, cyc_ref[...], preferred_element_type=jnp.float32)
    o_ref[...] = x_ref[...] + (sig + pe_ref[...])[None]


def kernel(x, yang_wheel, yin_wheel, grand_cycle_pe):
    b, s, d = x.shape
    assert s % S_TILE == 0 and d == DIM
    n_tiles = s // S_TILE

    cyc = _sc_gather_cyc(yang_wheel, yin_wheel)

    return pl.pallas_call(
        _enc_kernel,
        grid=(n_tiles, b),
        in_specs=[
            pl.BlockSpec((1, S_TILE, d), lambda i, j: (j, i, 0)),
            pl.BlockSpec((12, DIM), lambda i, j: (0, 0)),
            pl.BlockSpec((S_TILE, d), lambda i, j: (i, 0)),
        ],
        out_specs=pl.BlockSpec((1, S_TILE, d), lambda i, j: (j, i, 0)),
        out_shape=jax.ShapeDtypeStruct((b, s, d), x.dtype),
        compiler_params=pltpu.CompilerParams(
            dimension_semantics=("arbitrary", "arbitrary"),
        ),
    )(x, cyc, grand_cycle_pe)
